# baseline (speedup 1.0000x reference)
import jax
import jax.numpy as jnp
from jax import lax
from jax.experimental import pallas as pl
from jax.experimental.pallas import tpu as pltpu
from jax.experimental.pallas import tpu_sc as plsc

EPS = 0.1
TB = 25
NC, NS = 2, 16
RPW = 8


def _sc_select_body(dpad_hbm, u_hbm, l_hbm, dpad_v, u_v, l_v):
    wid = lax.axis_index("s") * NC + lax.axis_index("c")

    @pl.when(wid < 200 // RPW)
    def _():
        base = wid * RPW
        pltpu.sync_copy(dpad_hbm.at[pl.ds(base, RPW)], dpad_v)
        pltpu.sync_copy(u_hbm.at[pl.ds(base, RPW)], u_v)
        lane = lax.iota(jnp.int32, 16)
        lvec = jnp.zeros((16,), jnp.int32)
        for t in range(RPW):
            def chunk(c, carry):
                best, bidx = carry
                d16 = dpad_v[t, pl.ds(c * 16, 16)]
                u16 = u_v[t, pl.ds(c * 16, 16)]
                s = jnp.where(d16 != 1, u16, -1.0)
                idx = lax.iota(jnp.int32, 16) + c * 16
                better = s > best
                return jnp.where(better, s, best), jnp.where(better, idx, bidx)

            best, bidx = lax.fori_loop(
                0, 1024 // 16, chunk,
                (jnp.full((16,), -jnp.inf, jnp.float32),
                 jnp.zeros((16,), jnp.int32)))
            m = jnp.max(best)
            lval = jnp.min(jnp.where(best == m, bidx, 2**30))
            lvec = jnp.where(lane == t, lval, lvec)
        l_v[...] = lvec
        pltpu.sync_copy(l_v.at[pl.ds(0, RPW)], l_hbm.at[pl.ds(base, RPW)])


def _stream_body(l_sp, emb_ref, out_ref):
    i = pl.program_id(0)
    out_ref[...] = emb_ref[...]
    for t in range(TB):
        lt = l_sp[i * TB + t]
        row = emb_ref[t, pl.ds(lt, 1), :]
        norm = jnp.sqrt(jnp.sum(row * row, axis=1, keepdims=True))
        out_ref[t, pl.ds(lt, 1), :] = row + EPS * row / norm


def kernel(emb, data, dpadder, emb_matr):
    tlen, bz, d = emb.shape
    u = jax.random.uniform(jax.random.key(42), (tlen, bz))

    l = pl.kernel(
        _sc_select_body,
        out_type=jax.ShapeDtypeStruct((tlen,), jnp.int32),
        mesh=plsc.VectorSubcoreMesh(core_axis_name="c", subcore_axis_name="s"),
        compiler_params=pltpu.CompilerParams(needs_layout_passes=False),
        scratch_types=[
            pltpu.VMEM((RPW, bz), jnp.int32),
            pltpu.VMEM((RPW, bz), jnp.float32),
            pltpu.VMEM((16,), jnp.int32),
        ],
    )(dpadder, u)

    a = pl.pallas_call(
        _stream_body,
        grid_spec=pltpu.PrefetchScalarGridSpec(
            num_scalar_prefetch=1,
            grid=(tlen // TB,),
            in_specs=[pl.BlockSpec((TB, bz, d), lambda i, l_sp: (i, 0, 0))],
            out_specs=pl.BlockSpec((TB, bz, d), lambda i, l_sp: (i, 0, 0)),
        ),
        out_shape=jax.ShapeDtypeStruct((tlen, bz, d), emb.dtype),
    )(l, emb)
    return a, l


# speedup vs baseline: 1.1872x; 1.1872x over previous
import jax
import jax.numpy as jnp
from jax.experimental import pallas as pl
from jax.experimental.pallas import tpu as pltpu

EPS = 0.1
TB = 25


def _fused_body(emb_ref, dpad_ref, u_ref, out_ref, l_ref):
    scores = jnp.where(dpad_ref[0] != 1, u_ref[0], -1.0)
    m = jnp.max(scores, axis=1, keepdims=True)
    col = jax.lax.broadcasted_iota(jnp.int32, scores.shape, 1)
    l = jnp.min(jnp.where(scores == m, col, 2**30), axis=1, keepdims=True)
    l_ref[0] = jnp.broadcast_to(l, l_ref.shape[1:])

    row_iota = jax.lax.broadcasted_iota(
        jnp.int32, (emb_ref.shape[1], emb_ref.shape[2]), 0)
    for t in range(TB):
        emb_t = emb_ref[t]
        mask_t = row_iota == l[t:t + 1, 0:1]
        eword = jnp.max(jnp.where(mask_t, emb_t, -jnp.inf), axis=0, keepdims=True)
        norm = jnp.sqrt(jnp.sum(eword * eword, axis=1, keepdims=True))
        adv = eword + EPS * eword / norm
        out_ref[t] = jnp.where(mask_t, adv, emb_t)


def kernel(emb, data, dpadder, emb_matr):
    tlen, bz, d = emb.shape
    u = jax.random.uniform(jax.random.key(42), (tlen, bz))
    nb = tlen // TB
    dpad3 = dpadder.reshape(nb, TB, bz)
    u3 = u.reshape(nb, TB, bz)

    a, l_wide = pl.pallas_call(
        _fused_body,
        grid=(nb,),
        in_specs=[
            pl.BlockSpec((TB, bz, d), lambda i: (i, 0, 0)),
            pl.BlockSpec((1, TB, bz), lambda i: (i, 0, 0)),
            pl.BlockSpec((1, TB, bz), lambda i: (i, 0, 0)),
        ],
        out_specs=[
            pl.BlockSpec((TB, bz, d), lambda i: (i, 0, 0)),
            pl.BlockSpec((1, TB, 128), lambda i: (i, 0, 0)),
        ],
        out_shape=[
            jax.ShapeDtypeStruct((tlen, bz, d), emb.dtype),
            jax.ShapeDtypeStruct((nb, TB, 128), jnp.int32),
        ],
    )(emb, dpad3, u3)
    return a, l_wide[:, :, 0].reshape(tlen)
